# baseline (speedup 1.0000x reference)
import functools

import jax
import jax.numpy as jnp
from jax import lax
from jax.experimental import pallas as pl
from jax.experimental.pallas import tpu as pltpu
from jax.experimental.pallas import tpu_sc as plsc

B = 16384
V = 1000000
D = 64
P = 64

_NC = 2
_NS = 16
_NW = _NC * _NS
_B_PER_W = B // _NW


@functools.cache
def _make_sc_gather():
    mesh = plsc.VectorSubcoreMesh(core_axis_name="c", subcore_axis_name="s")

    @functools.partial(
        pl.kernel,
        mesh=mesh,
        out_type=jax.ShapeDtypeStruct((B * D,), jnp.float32),
        scratch_types=[
            pltpu.VMEM((_B_PER_W,), jnp.int32),
            pltpu.VMEM((D, 128), jnp.int32),
            pltpu.VMEM((D * 128,), jnp.float32),
            pltpu.SemaphoreType.DMA,
        ],
        compiler_params=pltpu.CompilerParams(use_tc_tiling_on_sc=False),
    )
    def gather_kernel(tblT_hbm, ids_hbm, out_hbm, ids_v, widx_v, stage_v,
                      sem):
        wid = lax.axis_index("s") * _NC + lax.axis_index("c")
        base = wid * _B_PER_W
        pltpu.sync_copy(ids_hbm.at[pl.ds(base, _B_PER_W)], ids_v)
        flat = tblT_hbm.at[0]
        for c in range(4):
            for s in range(8):
                v = ids_v[pl.ds(128 * c + 16 * s, 16)]
                for d in range(D):
                    widx_v[d, pl.ds(16 * s, 16)] = v + d * V
            copies = []
            for d in range(D):
                copies.append(pltpu.async_copy(
                    flat.at[widx_v.at[d]],
                    stage_v.at[pl.ds(d * 128, 128)],
                    sem))
            for cp in copies:
                cp.wait()
            pltpu.sync_copy(
                stage_v.at[pl.ds(0, 128 * D)],
                out_hbm.at[pl.ds((base + c * 128) * D, 128 * D)])

    return gather_kernel


def _dense_body(uet_ref, pft_ref, w1t_ref, b1_ref,
                w2t_ref, b2_ref, wfut_ref, wfpt_ref, bf_ref, out_ref):
    ht = jnp.maximum(
        jnp.dot(w1t_ref[...], pft_ref[...],
                preferred_element_type=jnp.float32) + b1_ref[...], 0.0)
    pet = (jnp.dot(w2t_ref[...], ht, preferred_element_type=jnp.float32)
           + b2_ref[...])
    acc = (jnp.dot(wfut_ref[...], uet_ref[...],
                   preferred_element_type=jnp.float32)
           + jnp.dot(wfpt_ref[...], pet, preferred_element_type=jnp.float32)
           + bf_ref[...])
    out_ref[...] = jnp.tanh(acc)


_BN = 2048


def _dense(uet, pft, W1t, b1c, W2t, b2c, Wfut, Wfpt, bfc):
    grid = (B // _BN,)

    def full(r, c):
        return pl.BlockSpec((r, c), lambda i: (0, 0))

    return pl.pallas_call(
        _dense_body,
        grid=grid,
        in_specs=[
            pl.BlockSpec((D, _BN), lambda i: (0, i)),
            pl.BlockSpec((P, _BN), lambda i: (0, i)),
            full(D // 2, P),
            full(D // 2, 1),
            full(D, D // 2),
            full(D, 1),
            full(D, D),
            full(D, D),
            full(D, 1),
        ],
        out_specs=pl.BlockSpec((D, _BN), lambda i: (0, i)),
        out_shape=jax.ShapeDtypeStruct((D, B), jnp.float32),
    )(uet, pft, W1t, b1c, W2t, b2c, Wfut, Wfpt, bfc)


def kernel(user_ids, profile_features, table, W1, b1, W2, b2, Wf, bf):
    ids = user_ids.astype(jnp.int32)

    out_flat = _make_sc_gather()(table.T, ids)
    uet = out_flat.reshape(_NW, 4, D, 128).transpose(2, 0, 1, 3)
    uet = uet.reshape(D, B)

    pft = profile_features.T
    out_t = _dense(
        uet, pft,
        W1.T, b1.reshape(-1, 1),
        W2.T, b2.reshape(-1, 1),
        Wf[:D].T, Wf[D:].T, bf.reshape(-1, 1),
    )
    return out_t.T


# speedup vs baseline: 21.8844x; 21.8844x over previous
import functools

import jax
import jax.numpy as jnp
from jax import lax
from jax.experimental import pallas as pl
from jax.experimental.pallas import tpu as pltpu
from jax.experimental.pallas import tpu_sc as plsc

B = 16384
V = 1000000
D = 64
P = 64

VT = 999936
NT = VT // 128
GW = NT * 1024

_NC = 2
_NS = 16
_NW = _NC * _NS
_B_PER_W = B // _NW
_CHUNK = 128
_NCHUNK = _B_PER_W // _CHUNK


@functools.cache
def _make_sc_gather():
    mesh = plsc.VectorSubcoreMesh(core_axis_name="c", subcore_axis_name="s")

    @functools.partial(
        pl.kernel,
        mesh=mesh,
        out_type=jax.ShapeDtypeStruct((B * D,), jnp.float32),
        scratch_types=[
            pltpu.VMEM((_B_PER_W,), jnp.int32),
            pltpu.VMEM((8, _CHUNK), jnp.int32),
            pltpu.VMEM((D * _CHUNK,), jnp.float32),
            pltpu.SemaphoreType.DMA,
        ],
        compiler_params=pltpu.CompilerParams(use_tc_tiling_on_sc=False),
    )
    def gather_kernel(g0, g1, g2, g3, g4, g5, g6, g7, ids_hbm,
                      out_hbm, ids_v, widx_v, stage_v, sem):
        groups = (g0, g1, g2, g3, g4, g5, g6, g7)
        wid = lax.axis_index("s") * _NC + lax.axis_index("c")
        base = wid * _B_PER_W
        pltpu.sync_copy(ids_hbm.at[pl.ds(base, _B_PER_W)], ids_v)
        for c in range(_NCHUNK):
            for s in range(8):
                v = ids_v[pl.ds(c * _CHUNK + 16 * s, 16)]
                vc = jnp.where(v >= VT, 0, v)
                wbase = (vc >> 7) * 1024 + (vc & 127)
                for j in range(8):
                    widx_v[j, pl.ds(16 * s, 16)] = wbase + 128 * j
            copies = []
            for g in range(8):
                for j in range(8):
                    copies.append(pltpu.async_copy(
                        groups[g].at[widx_v.at[j]],
                        stage_v.at[pl.ds((8 * g + j) * _CHUNK, _CHUNK)],
                        sem))
            for cp in copies:
                cp.wait()
            pltpu.sync_copy(
                stage_v,
                out_hbm.at[pl.ds((base + c * _CHUNK) * D, _CHUNK * D)])

    return gather_kernel


def _dense_body(uet_ref, oh_ref, remt_ref, pft_ref, w1t_ref, b1_ref,
                w2t_ref, b2_ref, wfut_ref, wfpt_ref, bf_ref, out_ref):
    oh = oh_ref[...]
    m = jnp.sum(oh, axis=0, keepdims=True)
    uet = (uet_ref[...] * (1.0 - m)
           + jnp.dot(remt_ref[...], oh, preferred_element_type=jnp.float32))
    ht = jnp.maximum(
        jnp.dot(w1t_ref[...], pft_ref[...],
                preferred_element_type=jnp.float32) + b1_ref[...], 0.0)
    pet = (jnp.dot(w2t_ref[...], ht, preferred_element_type=jnp.float32)
           + b2_ref[...])
    acc = (jnp.dot(wfut_ref[...], uet, preferred_element_type=jnp.float32)
           + jnp.dot(wfpt_ref[...], pet, preferred_element_type=jnp.float32)
           + bf_ref[...])
    out_ref[...] = jnp.tanh(acc)


_BN = 2048


def _dense(uet, oh, remt, pft, W1t, b1c, W2t, b2c, Wfut, Wfpt, bfc):
    grid = (B // _BN,)

    def full(r, c):
        return pl.BlockSpec((r, c), lambda i: (0, 0))

    return pl.pallas_call(
        _dense_body,
        grid=grid,
        in_specs=[
            pl.BlockSpec((D, _BN), lambda i: (0, i)),
            pl.BlockSpec((D, _BN), lambda i: (0, i)),
            full(D, D),
            pl.BlockSpec((P, _BN), lambda i: (0, i)),
            full(D // 2, P),
            full(D // 2, 1),
            full(D, D // 2),
            full(D, 1),
            full(D, D),
            full(D, D),
            full(D, 1),
        ],
        out_specs=pl.BlockSpec((D, _BN), lambda i: (0, i)),
        out_shape=jax.ShapeDtypeStruct((D, B), jnp.float32),
    )(uet, oh, remt, pft, W1t, b1c, W2t, b2c, Wfut, Wfpt, bfc)


def kernel(user_ids, profile_features, table, W1, b1, W2, b2, Wf, bf):
    ids = user_ids.astype(jnp.int32)

    groups = []
    for g in range(8):
        grp = lax.slice(table, (0, 8 * g), (VT, 8 * g + 8))
        g3 = grp.T.reshape(8, NT, 128).transpose(1, 0, 2)
        groups.append(g3.reshape(GW))

    out_flat = _make_sc_gather()(*groups, ids)
    uet = out_flat.reshape(B // _CHUNK, D, _CHUNK).transpose(1, 0, 2)
    uet = uet.reshape(D, B)

    oh = (jnp.arange(D, dtype=jnp.int32)[:, None]
          == (ids - VT)[None, :]).astype(jnp.float32)
    remt = table[VT:, :].T

    pft = profile_features.T
    out_t = _dense(
        uet, oh, remt, pft,
        W1.T, b1.reshape(-1, 1),
        W2.T, b2.reshape(-1, 1),
        Wf[:D].T, Wf[D:].T, bf.reshape(-1, 1),
    )
    return out_t.T
